# baseline (speedup 1.0000x reference)
import functools

import jax
import jax.numpy as jnp
from jax import lax
from jax.experimental import pallas as pl
from jax.experimental.pallas import tpu as pltpu
from jax.experimental.pallas import tpu_sc as plsc

VOCAB = 1_000_000
D = 128
SEQ = 50
NBATCH = 16384
NSPLIT = 2
NBS = NBATCH // NSPLIT
NW = 32
BPW = NBS // NW
CB = 2
NCHUNK = BPW // CB
NB = 4
A = 3
NOUTER = NCHUNK // NB
assert NOUTER * NB == NCHUNK


def _gather_kernel(table_hbm, idx_hbm, out_hbm, idx_full, rows_v, *sems):
    gsems = sems[:NB]
    osems = sems[NB:]
    wid = lax.axis_index("s") * 2 + lax.axis_index("c")
    row0 = wid * BPW

    pltpu.sync_copy(idx_hbm.at[wid], idx_full)

    def start_gather(g, b):
        pltpu.make_async_copy(table_hbm.at[idx_full.at[g]], rows_v.at[b],
                              gsems[b]).start()

    def wait_gather(g, b):
        pltpu.make_async_copy(table_hbm.at[idx_full.at[g]], rows_v.at[b],
                              gsems[b]).wait()

    def start_out(g, b):
        for c in range(CB):
            pltpu.make_async_copy(rows_v.at[b, pl.ds(c * SEQ, SEQ)],
                                  out_hbm.at[row0 + g * CB + c],
                                  osems[b]).start()

    def wait_out(g, b):
        for c in range(CB):
            pltpu.make_async_copy(rows_v.at[b, pl.ds(c * SEQ, SEQ)],
                                  out_hbm.at[row0 + g * CB + c],
                                  osems[b]).wait()

    for g in range(A):
        start_gather(g, g % NB)

    def body(t, carry):
        for b0 in range(NB):
            g = t * NB + b0
            wait_gather(g, b0)
            start_out(g, b0)
            ga = g + A
            ba = (b0 + A) % NB

            @pl.when(ga < NCHUNK)
            def _issue_ahead():
                @pl.when(ga >= NB)
                def _reuse_guard():
                    wait_out(ga - NB, ba)
                start_gather(ga, ba)
        return carry

    lax.fori_loop(0, NOUTER, body, 0)

    for b in range(NB):
        g = NCHUNK - NB + b
        wait_out(g, g % NB)


def kernel(indices, embeddings):
    idx_all = indices.reshape(NSPLIT, NW, NCHUNK, CB * SEQ).astype(jnp.int32)
    mesh = plsc.VectorSubcoreMesh(core_axis_name="c", subcore_axis_name="s")
    run = functools.partial(
        pl.kernel,
        mesh=mesh,
        out_type=jax.ShapeDtypeStruct((NBS, SEQ, D), jnp.float32),
        compiler_params=pltpu.CompilerParams(use_tc_tiling_on_sc=True),
        scratch_types=[
            pltpu.VMEM((NCHUNK, CB * SEQ), jnp.int32),
            pltpu.VMEM((NB, CB * SEQ, D), jnp.float32),
        ] + [pltpu.SemaphoreType.DMA] * (2 * NB),
    )(_gather_kernel)
    out = jnp.zeros((NBATCH, SEQ, D), jnp.float32)
    for s in range(NSPLIT):
        out = lax.dynamic_update_slice(out, run(embeddings, idx_all[s]),
                                       (s * NBS, 0, 0))
    return out


# speedup vs baseline: 1.7178x; 1.7178x over previous
import functools

import jax
import jax.numpy as jnp
from jax import lax
from jax.experimental import pallas as pl
from jax.experimental.pallas import tpu as pltpu
from jax.experimental.pallas import tpu_sc as plsc

VOCAB = 1_000_000
D = 128
SEQ = 50
NBATCH = 16384
NW = 32
BPW = NBATCH // NW
CB = 2
NCHUNK = BPW // CB
NB = 4
A = 3
NOUTER = NCHUNK // NB
assert NOUTER * NB == NCHUNK


def _gather_kernel(table_hbm, idx_hbm, out_hbm, idx_full, rows_v, *sems):
    gsems = sems[:NB]
    osems = sems[NB:]
    wid = lax.axis_index("s") * 2 + lax.axis_index("c")
    row0 = wid * BPW

    pltpu.sync_copy(idx_hbm.at[wid], idx_full)

    def start_gather(g, b):
        pltpu.make_async_copy(table_hbm.at[idx_full.at[g]], rows_v.at[b],
                              gsems[b]).start()

    def wait_gather(g, b):
        pltpu.make_async_copy(table_hbm.at[idx_full.at[g]], rows_v.at[b],
                              gsems[b]).wait()

    def start_out(g, b):
        for c in range(CB):
            pltpu.make_async_copy(rows_v.at[b, pl.ds(c * SEQ, SEQ)],
                                  out_hbm.at[row0 + g * CB + c],
                                  osems[b]).start()

    def wait_out(g, b):
        for c in range(CB):
            pltpu.make_async_copy(rows_v.at[b, pl.ds(c * SEQ, SEQ)],
                                  out_hbm.at[row0 + g * CB + c],
                                  osems[b]).wait()

    for g in range(A):
        start_gather(g, g % NB)

    def body(t, carry):
        for b0 in range(NB):
            g = t * NB + b0
            wait_gather(g, b0)
            start_out(g, b0)
            ga = g + A
            ba = (b0 + A) % NB

            @pl.when(ga < NCHUNK)
            def _issue_ahead():
                @pl.when(ga >= NB)
                def _reuse_guard():
                    wait_out(ga - NB, ba)
                start_gather(ga, ba)
        return carry

    lax.fori_loop(0, NOUTER, body, 0)

    for b in range(NB):
        g = NCHUNK - NB + b
        wait_out(g, g % NB)


def kernel(indices, embeddings):
    idx = indices.reshape(NW, NCHUNK, CB * SEQ).astype(jnp.int32)
    mesh = plsc.VectorSubcoreMesh(core_axis_name="c", subcore_axis_name="s")
    run = functools.partial(
        pl.kernel,
        mesh=mesh,
        out_type=jax.ShapeDtypeStruct((NBATCH, SEQ, D), jnp.float32),
        compiler_params=pltpu.CompilerParams(use_tc_tiling_on_sc=True),
        scratch_types=[
            pltpu.VMEM((NCHUNK, CB * SEQ), jnp.int32),
            pltpu.VMEM((NB, CB * SEQ, D), jnp.float32),
        ] + [pltpu.SemaphoreType.DMA] * (2 * NB),
    )(_gather_kernel)
    return run(embeddings, idx)
